# baseline (speedup 1.0000x reference)
import functools

import jax
import jax.numpy as jnp
from jax import lax
from jax.experimental import pallas as pl
from jax.experimental.pallas import tpu as pltpu
from jax.experimental.pallas import tpu_sc as plsc

_VOCAB = 1000000
_DIM = 64
_SEQ = 200
_BATCH = 4096
_NC, _NS = 2, 16
_NW = _NC * _NS
_COLS = _BATCH // _NW
_NPAIR = _SEQ // 2
_LANES = 16
_NBLK = _COLS // _LANES

_mesh = plsc.VectorSubcoreMesh(core_axis_name="c", subcore_axis_name="s")


@functools.partial(
    pl.kernel,
    mesh=_mesh,
    out_type=jax.ShapeDtypeStruct((_SEQ, _DIM, _BATCH), jnp.float32),
    scratch_types=[
        pltpu.VMEM((_SEQ, _COLS), jnp.int32),
        pltpu.VMEM((_SEQ, _COLS), jnp.int32),
        pltpu.VMEM((_COLS, 128), jnp.float32),
        pltpu.VMEM((_COLS, 128), jnp.float32),
        pltpu.VMEM((_DIM, _COLS), jnp.float32),
        pltpu.VMEM((_DIM, _COLS), jnp.float32),
        pltpu.SemaphoreType.DMA,
        pltpu.SemaphoreType.DMA,
        pltpu.SemaphoreType.DMA,
        pltpu.SemaphoreType.DMA,
    ],
    compiler_params=pltpu.CompilerParams(use_tc_tiling_on_sc=True,
                                         needs_layout_passes=False),
)
def _emb_lookup(idx_hbm, table_hbm, out_hbm, idx_v, pid_v,
                pairs0, pairs1, trows0, trows1,
                gsem0, gsem1, osem0, osem1):
    wid = lax.axis_index("s") * _NC + lax.axis_index("c")
    col = pl.multiple_of(wid * _COLS, _COLS)
    pairs = (pairs0, pairs1)
    trows = (trows0, trows1)
    gsem = (gsem0, gsem1)
    osem = (osem0, osem1)

    pltpu.sync_copy(idx_hbm.at[pl.ds(0, _SEQ), pl.ds(col, _COLS)], idx_v)

    def halve(s, carry):
        for blk in range(_NBLK):
            v = idx_v[s, pl.ds(blk * _LANES, _LANES)]
            pid_v[s, pl.ds(blk * _LANES, _LANES)] = lax.shift_right_logical(v, 1)
        return carry

    lax.fori_loop(0, _SEQ, halve, 0)

    biota = lax.iota(jnp.int32, _LANES)

    def fire(s, b):
        pltpu.async_copy(table_hbm.at[pid_v.at[s]], pairs[b], gsem[b])

    def drain_gather(b):
        pltpu.make_async_copy(table_hbm.at[pl.ds(0, _COLS)], pairs[b],
                              gsem[b]).wait()

    def extract(s, b):
        bases = []
        for blk in range(_NBLK):
            v = idx_v[s, pl.ds(blk * _LANES, _LANES)]
            par = lax.shift_left(lax.rem(v, 2), 6)
            rows16 = biota + blk * _LANES
            bases.append((rows16, par))

        @plsc.parallel_loop(0, _DIM, step=8)
        def dchunk(d0):
            for blk in range(_NBLK):
                rows16, par = bases[blk]
                base = par + d0
                vals = [plsc.load_gather(pairs[b], [rows16, base + dd])
                        for dd in range(8)]
                for dd in range(8):
                    trows[b][d0 + dd, pl.ds(blk * _LANES, _LANES)] = vals[dd]

    def out_start(s, b):
        pltpu.async_copy(trows[b],
                         out_hbm.at[s, pl.ds(0, _DIM), pl.ds(col, _COLS)],
                         osem[b])

    def drain_out(b):
        pltpu.make_async_copy(trows[b],
                              out_hbm.at[0, pl.ds(0, _DIM), pl.ds(0, _COLS)],
                              osem[b]).wait()

    fire(0, 0)
    drain_gather(0)
    fire(1, 1)
    extract(0, 0)
    out_start(0, 0)
    drain_gather(1)
    fire(2, 0)
    extract(1, 1)
    out_start(1, 1)

    def body(t, carry):
        s0 = 2 * t
        drain_gather(0)
        fire(s0 + 1, 1)
        drain_out(0)
        extract(s0, 0)
        out_start(s0, 0)
        drain_gather(1)
        fire(s0 + 2, 0)
        drain_out(1)
        extract(s0 + 1, 1)
        out_start(s0 + 1, 1)
        return carry

    lax.fori_loop(1, _NPAIR - 1, body, 0)

    drain_gather(0)
    fire(_SEQ - 1, 1)
    drain_out(0)
    extract(_SEQ - 2, 0)
    out_start(_SEQ - 2, 0)
    drain_gather(1)
    drain_out(1)
    extract(_SEQ - 1, 1)
    out_start(_SEQ - 1, 1)
    drain_out(0)
    drain_out(1)


def kernel(src_input, word_lut):
    idx = src_input.reshape(_SEQ, _BATCH)
    table_pairs = word_lut.reshape(_VOCAB // 2, 2 * _DIM)
    out_t = _emb_lookup(idx, table_pairs)
    return jnp.swapaxes(out_t, 1, 2)


# speedup vs baseline: 1.1080x; 1.1080x over previous
import functools

import jax
import jax.numpy as jnp
from jax import lax
from jax.experimental import pallas as pl
from jax.experimental.pallas import tpu as pltpu
from jax.experimental.pallas import tpu_sc as plsc

_VOCAB = 1000000
_DIM = 64
_SEQ = 200
_BATCH = 4096

_NC, _NS = 2, 16
_NW = _NC * _NS
_COLS = _BATCH // _NW
_G = 5
_NCHUNK = _SEQ // _G
_NPAIR = _NCHUNK // 2

_mesh = plsc.VectorSubcoreMesh(core_axis_name="c", subcore_axis_name="s")


@functools.partial(
    pl.kernel,
    mesh=_mesh,
    out_type=jax.ShapeDtypeStruct((_SEQ, _BATCH, _DIM), jnp.float32),
    scratch_types=[
        pltpu.VMEM((_SEQ, _COLS), jnp.int32),
        pltpu.VMEM((_G, _COLS, _DIM), jnp.float32),
        pltpu.VMEM((_G, _COLS, _DIM), jnp.float32),
        pltpu.SemaphoreType.DMA,
        pltpu.SemaphoreType.DMA,
        pltpu.SemaphoreType.DMA,
        pltpu.SemaphoreType.DMA,
    ],
    compiler_params=pltpu.CompilerParams(use_tc_tiling_on_sc=False),
)
def _emb_lookup(idx_hbm, table_hbm, out_hbm, idx_v, rows0, rows1,
                gsem0, gsem1, osem0, osem1):
    wid = lax.axis_index("s") * _NC + lax.axis_index("c")
    col = pl.multiple_of(wid * _COLS, _COLS)
    rows = (rows0, rows1)
    gsem = (gsem0, gsem1)
    osem = (osem0, osem1)

    pltpu.sync_copy(idx_hbm.at[pl.ds(0, _SEQ), pl.ds(col, _COLS)], idx_v)

    def fire(c, b):
        for g in range(_G):
            pltpu.async_copy(
                table_hbm.at[idx_v.at[c * _G + g]],
                rows[b].at[g],
                gsem[b],
            )

    def drain_gather(b):
        pltpu.make_async_copy(
            out_hbm.at[pl.ds(0, _G), pl.ds(0, _COLS)], rows[b], gsem[b]
        ).wait()

    def out_start(c, b):
        pltpu.async_copy(
            rows[b],
            out_hbm.at[pl.ds(c * _G, _G), pl.ds(col, _COLS)],
            osem[b],
        )

    def drain_out(b):
        pltpu.make_async_copy(
            rows[b], out_hbm.at[pl.ds(0, _G), pl.ds(0, _COLS)], osem[b]
        ).wait()

    fire(0, 0)
    drain_gather(0)
    out_start(0, 0)
    fire(1, 1)
    drain_gather(1)
    out_start(1, 1)
    drain_out(0)
    fire(2, 0)

    def body(t, carry):
        c0 = 2 * t
        drain_gather(0)
        out_start(c0, 0)
        drain_out(1)
        fire(c0 + 1, 1)
        drain_gather(1)
        out_start(c0 + 1, 1)
        drain_out(0)
        fire(c0 + 2, 0)
        return carry

    lax.fori_loop(1, _NPAIR - 1, body, 0)

    drain_gather(0)
    out_start(_NCHUNK - 2, 0)
    drain_out(1)
    fire(_NCHUNK - 1, 1)
    drain_gather(1)
    out_start(_NCHUNK - 1, 1)
    drain_out(0)
    drain_out(1)


def kernel(src_input, word_lut):
    idx = src_input.reshape(_SEQ, _BATCH)
    return _emb_lookup(idx, word_lut)
